# baseline (speedup 1.0000x reference)
import dataclasses
import functools

import jax
import jax.numpy as jnp
from jax import lax
from jax.experimental import pallas as pl
from jax.experimental.pallas import tpu as pltpu
from jax.experimental.pallas import tpu_sc as plsc

N_WORKERS = 32
LANES = 16
D_FEAT = 128
CHUNK = 368


@functools.cache
def _edge_dot_fn(E: int):
    epw = E // N_WORKERS
    n_chunks = epw // CHUNK
    assert epw % CHUNK == 0 and CHUNK % LANES == 0 and epw % 8 == 0

    mesh = plsc.VectorSubcoreMesh(core_axis_name="c", subcore_axis_name="s")

    cp = pltpu.CompilerParams()
    if "needs_layout_passes" in pltpu.CompilerParams.__dataclass_fields__:
        cp = dataclasses.replace(cp, needs_layout_passes=False)

    @functools.partial(
        pl.kernel,
        compiler_params=cp,
        out_type=jax.ShapeDtypeStruct((E,), jnp.float32),
        mesh=mesh,
        scratch_types=[
            pltpu.VMEM((2 * epw,), jnp.int32),
            pltpu.VMEM((2 * CHUNK, D_FEAT), jnp.float32),
            pltpu.VMEM((epw,), jnp.float32),
            pltpu.VMEM((LANES, LANES), jnp.float32),
            pltpu.SemaphoreType.DMA,
        ],
    )
    def kern(h_hbm, idx_hbm, out_hbm,
             cidx, rows_v, out_v, acc_v, sem_g):
        wid = lax.axis_index("s") * 2 + lax.axis_index("c")
        base = wid * epw

        pltpu.sync_copy(idx_hbm.at[pl.ds(2 * base, 2 * epw)], cidx)

        @pl.loop(0, n_chunks)
        def _(j):
            off = j * CHUNK
            cg = pltpu.async_copy(
                h_hbm.at[cidx.at[pl.ds(2 * off, 2 * CHUNK)]], rows_v, sem_g)
            cg.wait()

            @pl.loop(0, CHUNK, step=LANES)
            def _(g):
                for e in range(LANES):
                    r = 2 * (g + e)
                    a = (rows_v[r, pl.ds(0, LANES)]
                         * rows_v[r + 1, pl.ds(0, LANES)])
                    for s_ in range(1, D_FEAT // LANES):
                        a += (rows_v[r, pl.ds(s_ * LANES, LANES)]
                              * rows_v[r + 1, pl.ds(s_ * LANES, LANES)])
                    acc_v[e] = a
                rows_i = lax.iota(jnp.int32, LANES)
                s_vec = jnp.zeros((LANES,), jnp.float32)
                for f in range(LANES):
                    cols_i = jnp.full((LANES,), f, jnp.int32)
                    s_vec += plsc.load_gather(acc_v, [rows_i, cols_i])
                out_v[pl.ds(off + g, LANES)] = s_vec

        pltpu.sync_copy(out_v, out_hbm.at[pl.ds(base, epw)])

    return kern


def kernel(h, edge_index):
    E = edge_index.shape[1]
    step = N_WORKERS * CHUNK
    E_pad = ((E + step - 1) // step) * step
    src = edge_index[0].astype(jnp.int32)
    dst = edge_index[1].astype(jnp.int32)
    if E_pad != E:
        src = jnp.pad(src, (0, E_pad - E))
        dst = jnp.pad(dst, (0, E_pad - E))
    cidx = jnp.stack([src, dst], axis=1).reshape(-1)
    out = _edge_dot_fn(E_pad)(h, cidx)
    return out[:E] if E_pad != E else out


# speedup vs baseline: 4.1025x; 4.1025x over previous
import dataclasses
import functools

import jax
import jax.numpy as jnp
from jax import lax
from jax.experimental import pallas as pl
from jax.experimental.pallas import tpu as pltpu
from jax.experimental.pallas import tpu_sc as plsc

N_WORKERS = 32
LANES = 16
D_FEAT = 128
CHUNK = 400
HALF = CHUNK // 2


@functools.cache
def _edge_dot_fn(E: int):
    epw = E // N_WORKERS
    n_chunks = epw // CHUNK
    assert epw % CHUNK == 0 and CHUNK % LANES == 0 and epw % 8 == 0

    mesh = plsc.VectorSubcoreMesh(core_axis_name="c", subcore_axis_name="s")

    cp = pltpu.CompilerParams()
    if "needs_layout_passes" in pltpu.CompilerParams.__dataclass_fields__:
        cp = dataclasses.replace(cp, needs_layout_passes=False)

    @functools.partial(
        pl.kernel,
        compiler_params=cp,
        out_type=jax.ShapeDtypeStruct((E,), jnp.float32),
        mesh=mesh,
        scratch_types=[
            pltpu.VMEM((epw,), jnp.int32),
            pltpu.VMEM((epw,), jnp.int32),
            pltpu.VMEM((CHUNK, D_FEAT), jnp.float32),
            pltpu.VMEM((CHUNK, D_FEAT), jnp.float32),
            pltpu.VMEM((CHUNK,), jnp.float32),
            pltpu.VMEM((LANES, LANES), jnp.float32),
            pltpu.SemaphoreType.DMA,
            pltpu.SemaphoreType.DMA,
            pltpu.SemaphoreType.DMA,
            pltpu.SemaphoreType.DMA,
        ],
    )
    def kern(h_hbm, src_hbm, dst_hbm, out_hbm,
             sidx, didx, u_v, v_v, out_v, acc_v, s0, s1, s2, s3):
        wid = lax.axis_index("s") * 2 + lax.axis_index("c")
        base = wid * epw

        pltpu.sync_copy(src_hbm.at[pl.ds(base, epw)], sidx)
        pltpu.sync_copy(dst_hbm.at[pl.ds(base, epw)], didx)

        @pl.loop(0, n_chunks)
        def _(j):
            off = j * CHUNK
            copies = (
                pltpu.async_copy(
                    h_hbm.at[sidx.at[pl.ds(off, HALF)]],
                    u_v.at[pl.ds(0, HALF)], s0),
                pltpu.async_copy(
                    h_hbm.at[sidx.at[pl.ds(off + HALF, HALF)]],
                    u_v.at[pl.ds(HALF, HALF)], s1),
                pltpu.async_copy(
                    h_hbm.at[didx.at[pl.ds(off, HALF)]],
                    v_v.at[pl.ds(0, HALF)], s2),
                pltpu.async_copy(
                    h_hbm.at[didx.at[pl.ds(off + HALF, HALF)]],
                    v_v.at[pl.ds(HALF, HALF)], s3),
            )
            for c in copies:
                c.wait()

            @pl.loop(0, CHUNK, step=LANES)
            def _(g):
                for e in range(LANES):
                    a = (u_v[g + e, pl.ds(0, LANES)]
                         * v_v[g + e, pl.ds(0, LANES)])
                    for s_ in range(1, D_FEAT // LANES):
                        a += (u_v[g + e, pl.ds(s_ * LANES, LANES)]
                              * v_v[g + e, pl.ds(s_ * LANES, LANES)])
                    acc_v[e] = a
                rows_i = lax.iota(jnp.int32, LANES)
                s_vec = jnp.zeros((LANES,), jnp.float32)
                for f in range(LANES):
                    cols_i = jnp.full((LANES,), f, jnp.int32)
                    s_vec += plsc.load_gather(acc_v, [rows_i, cols_i])
                out_v[pl.ds(g, LANES)] = s_vec

            pltpu.sync_copy(out_v, out_hbm.at[pl.ds(base + off, CHUNK)])

    return kern


def kernel(h, edge_index):
    E = edge_index.shape[1]
    step = N_WORKERS * CHUNK
    E_pad = ((E + step - 1) // step) * step
    src = edge_index[0].astype(jnp.int32)
    dst = edge_index[1].astype(jnp.int32)
    if E_pad != E:
        src = jnp.pad(src, (0, E_pad - E))
        dst = jnp.pad(dst, (0, E_pad - E))
    out = _edge_dot_fn(E_pad)(h, src, dst)
    return out[:E] if E_pad != E else out
